# baseline (speedup 1.0000x reference)
import functools

import jax
import jax.numpy as jnp
import numpy as np
from jax import lax
from jax.experimental import pallas as pl
from jax.experimental.pallas import tpu as pltpu
from jax.experimental.pallas import tpu_sc as plsc

_NC, _NS, _L = 2, 16, 16
_NW = _NC * _NS

_CHUNK = 40
_NPHASE = 10


def _pack_bf16_pairs(v):
    b = lax.bitcast_convert_type(v, jnp.int32) + jnp.int32(0x8000)
    d2 = v.shape[1] // 2
    lo = lax.shift_right_logical(b[:, :d2], 16)
    hi = jnp.bitwise_and(b[:, d2:], jnp.int32(-65536))
    return jnp.bitwise_or(lo, hi)


def _t_body(x_ref, w_ref, b_ref, t_ref):
    d = x_ref.shape[1]
    xv = x_ref[...]
    u = jnp.exp(
        -(jnp.dot(xv, w_ref[...], preferred_element_type=jnp.float32) + b_ref[...])
    )
    t_ref[:, : d // 2] = _pack_bf16_pairs(xv)
    t_ref[:, d // 2 :] = _pack_bf16_pairs(u)


def _q_body(ea_ref, w_ref, q_ref):
    q_ref[...] = _pack_bf16_pairs(
        jnp.exp(
            -jnp.dot(ea_ref[...], w_ref[...], preferred_element_type=jnp.float32)
        )
    )


def _out_body(x_ref, a0_ref, a1_ref, wn_ref, wm_ref, bn_ref, bm_ref, o_ref):
    acc = jnp.dot(x_ref[...], wn_ref[...], preferred_element_type=jnp.float32)
    acc = acc + jnp.dot(
        a0_ref[0] + a1_ref[0], wm_ref[...], preferred_element_type=jnp.float32
    )
    o_ref[...] = jnp.tanh(acc + bn_ref[...] + bm_ref[...])


def kernel(x, edge_index, edge_attr, W_node, b_node, W_neigh, b_neigh, W_att, b_att):
    N, D = x.shape
    E, DE = edge_attr.shape
    row = edge_index[0].astype(jnp.int32)
    col = edge_index[1].astype(jnp.int32)
    W_att_x = W_att[:D]
    W_att_e = W_att[D:]

    BN = 1000
    T = pl.pallas_call(
        _t_body,
        grid=(N // BN,),
        in_specs=[
            pl.BlockSpec((BN, D), lambda i: (i, 0)),
            pl.BlockSpec((D, D), lambda i: (0, 0)),
            pl.BlockSpec((1, D), lambda i: (0, 0)),
        ],
        out_specs=pl.BlockSpec((BN, D), lambda i: (i, 0)),
        out_shape=jax.ShapeDtypeStruct((N, D), jnp.int32),
    )(x, W_att_x, b_att.reshape(1, D))

    BE = 2000
    Q = pl.pallas_call(
        _q_body,
        grid=(E // BE,),
        in_specs=[
            pl.BlockSpec((BE, DE), lambda i: (i, 0)),
            pl.BlockSpec((DE, D), lambda i: (0, 0)),
        ],
        out_specs=pl.BlockSpec((BE, D // 2), lambda i: (i, 0)),
        out_shape=jax.ShapeDtypeStruct((E, D // 2), jnp.int32),
    )(edge_attr, W_att_e)

    C = _CHUNK
    EPW = E // _NW
    NPHASE = _NPHASE
    CPP = EPW // C // NPHASE
    PAIRS = (CPP - 1) // 2
    RPT = ((N // _NS) + 7) // 8 * 8
    NPAD = RPT * _NS
    mesh = plsc.VectorSubcoreMesh(core_axis_name="c", subcore_axis_name="s")

    @functools.partial(
        pl.kernel,
        out_type=jax.ShapeDtypeStruct((_NC, NPAD, D), jnp.float32),
        mesh=mesh,
        scratch_types=[
            pltpu.VMEM((CPP, C), jnp.int32),
            pltpu.VMEM((CPP, C), jnp.int32),
            pltpu.VMEM((C, D), jnp.int32),
            pltpu.VMEM((C, D), jnp.int32),
            pltpu.VMEM((C, D // 2), jnp.int32),
            pltpu.VMEM((C, D // 2), jnp.int32),
            pltpu.VMEM((C, D), jnp.float32),
            pltpu.VMEM((C, D), jnp.float32),
            pltpu.MemorySpace.VMEM_SHARED((NPAD, D), jnp.float32),
            pltpu.SemaphoreType.DMA,
            pltpu.SemaphoreType.DMA,
            pltpu.SemaphoreType.DMA,
            pltpu.SemaphoreType.DMA,
            pltpu.SemaphoreType.DMA,
            pltpu.SemaphoreType.DMA,
        ],
    )
    def _sc_agg(t_hbm, q_hbm, col_hbm, row_hbm, z_hbm, out_hbm,
                colb, rowb, trA, trB, qA, qB, nA, nB, acc_sh,
                sga, sgb, sqa, sqb, ssa, ssb):
        cid = lax.axis_index("c")
        sid = lax.axis_index("s")
        wid = sid * _NC + cid
        pltpu.sync_copy(z_hbm, acc_sh.at[pl.ds(sid * RPT, RPT)])
        plsc.subcore_barrier()

        def issue(pbase, k, tr, q, sg, sq):
            pltpu.async_copy(t_hbm.at[colb.at[k]], tr, sg)
            pltpu.async_copy(q_hbm.at[pl.ds(pbase + k * C, C)], q, sq)

        def consume(pbase, k, tr, q, n, sg, sq, ss, wait_prev):
            pltpu.make_async_copy(t_hbm.at[colb.at[k]], tr, sg).wait()
            pltpu.make_async_copy(q_hbm.at[pl.ds(pbase + k * C, C)], q, sq).wait()

            def unpk(w):
                a = lax.bitcast_convert_type(
                    lax.shift_left(w, 16), jnp.float32)
                b = lax.bitcast_convert_type(
                    jnp.bitwise_and(w, jnp.int32(-65536)), jnp.float32)
                return a, b

            def sig_mul(xx, uu, vv):
                den = 1.0 + uu * vv
                seed = lax.bitcast_convert_type(
                    jnp.int32(0x7EF127EA)
                    - lax.bitcast_convert_type(den, jnp.int32),
                    jnp.float32,
                )
                r = seed * (2.0 - den * seed)
                r = r * (2.0 - den * r)
                return xx * r

            def sig_mul1(xx, uu, vv):
                den = 1.0 + uu * vv
                seed = lax.bitcast_convert_type(
                    jnp.int32(0x7EF127EA)
                    - lax.bitcast_convert_type(den, jnp.int32),
                    jnp.float32,
                )
                r = seed * (2.0 - den * seed)
                return xx * r

            def edge_body(e, carry2):
                h = D // 2
                for j in range(D // (2 * _L)):
                    xa, xb = unpk(tr[e, pl.ds(j * _L, _L)])
                    ua, ub = unpk(tr[e, pl.ds(h + j * _L, _L)])
                    va, vb = unpk(q[e, pl.ds(j * _L, _L)])
                    n[e, pl.ds(j * _L, _L)] = sig_mul1(xa, ua, va)
                    n[e, pl.ds(h + j * _L, _L)] = sig_mul1(xb, ub, vb)
                return carry2

            lax.fori_loop(0, C, edge_body, 0, unroll=16)
            pltpu.sync_copy(n, acc_sh.at[rowb.at[k]], add=True)

        def phase_body(p, carry):
            pltpu.sync_copy(col_hbm.at[wid, p], colb)
            pltpu.sync_copy(row_hbm.at[wid, p], rowb)
            pbase = (wid * NPHASE + p) * CPP * C
            issue(pbase, 0, trA, qA, sga, sqa)

            def pair_body(i, carry2):
                a = 2 * i
                issue(pbase, a + 1, trB, qB, sgb, sqb)
                consume(pbase, a, trA, qA, nA, sga, sqa, ssa, wait_prev=True)
                issue(pbase, a + 2, trA, qA, sga, sqa)
                consume(pbase, a + 1, trB, qB, nB, sgb, sqb, ssb, wait_prev=True)
                return carry2

            issue(pbase, 1, trB, qB, sgb, sqb)
            consume(pbase, 0, trA, qA, nA, sga, sqa, ssa, wait_prev=False)
            issue(pbase, 2, trA, qA, sga, sqa)
            consume(pbase, 1, trB, qB, nB, sgb, sqb, ssb, wait_prev=False)
            lax.fori_loop(1, PAIRS, pair_body, 0, unroll=False)
            consume(pbase, CPP - 1, trA, qA, nA, sga, sqa, ssa, wait_prev=True)
            return carry

        lax.fori_loop(0, NPHASE, phase_body, 0, unroll=False)

        plsc.subcore_barrier()
        pltpu.sync_copy(
            acc_sh.at[pl.ds(sid * RPT, RPT)],
            out_hbm.at[cid, pl.ds(sid * RPT, RPT)],
        )

    zeros = jnp.zeros((RPT, D), jnp.float32)
    col4 = col.reshape(_NW, NPHASE, CPP, C)
    row4 = row.reshape(_NW, NPHASE, CPP, C)
    agg2 = _sc_agg(T, Q, col4, row4, zeros)

    out = pl.pallas_call(
        _out_body,
        grid=(N // BN,),
        in_specs=[
            pl.BlockSpec((BN, D), lambda i: (i, 0)),
            pl.BlockSpec((1, BN, D), lambda i: (0, i, 0)),
            pl.BlockSpec((1, BN, D), lambda i: (1, i, 0)),
            pl.BlockSpec((D, D), lambda i: (0, 0)),
            pl.BlockSpec((D, D), lambda i: (0, 0)),
            pl.BlockSpec((1, D), lambda i: (0, 0)),
            pl.BlockSpec((1, D), lambda i: (0, 0)),
        ],
        out_specs=pl.BlockSpec((BN, D), lambda i: (i, 0)),
        out_shape=jax.ShapeDtypeStruct((N, D), jnp.float32),
    )(x, agg2, agg2, W_node, W_neigh, b_node.reshape(1, D), b_neigh.reshape(1, D))
    return out


# speedup vs baseline: 1.0994x; 1.0994x over previous
import functools

import jax
import jax.numpy as jnp
import numpy as np
from jax import lax
from jax.experimental import pallas as pl
from jax.experimental.pallas import tpu as pltpu
from jax.experimental.pallas import tpu_sc as plsc

_NC, _NS, _L = 2, 16, 16
_NW = _NC * _NS

_CHUNK = 40
_NPHASE = 10


def _pack_bf16_pairs(v):
    b = lax.bitcast_convert_type(v, jnp.int32) + jnp.int32(0x8000)
    d2 = v.shape[1] // 2
    lo = lax.shift_right_logical(b[:, :d2], 16)
    hi = jnp.bitwise_and(b[:, d2:], jnp.int32(-65536))
    return jnp.bitwise_or(lo, hi)


def _tq_body(nbt, x_ref, wx_ref, b_ref, ea_ref, we_ref, t_ref, q_ref):
    i = pl.program_id(0)

    @pl.when(i < nbt)
    def _():
        d = x_ref.shape[1]
        xv = x_ref[...]
        u = jnp.exp(
            -(jnp.dot(xv, wx_ref[...], preferred_element_type=jnp.float32)
              + b_ref[...])
        )
        t_ref[:, : d // 2] = _pack_bf16_pairs(xv)
        t_ref[:, d // 2 :] = _pack_bf16_pairs(u)

    @pl.when(i >= nbt)
    def _():
        q_ref[...] = _pack_bf16_pairs(
            jnp.exp(
                -jnp.dot(ea_ref[...], we_ref[...],
                         preferred_element_type=jnp.float32)
            )
        )


def _out_body(x_ref, a0_ref, a1_ref, wn_ref, wm_ref, bn_ref, bm_ref, o_ref):
    acc = jnp.dot(x_ref[...], wn_ref[...], preferred_element_type=jnp.float32)
    acc = acc + jnp.dot(
        a0_ref[0] + a1_ref[0], wm_ref[...], preferred_element_type=jnp.float32
    )
    o_ref[...] = jnp.tanh(acc + bn_ref[...] + bm_ref[...])


def kernel(x, edge_index, edge_attr, W_node, b_node, W_neigh, b_neigh, W_att, b_att):
    N, D = x.shape
    E, DE = edge_attr.shape
    row = edge_index[0].astype(jnp.int32)
    col = edge_index[1].astype(jnp.int32)
    W_att_x = W_att[:D]
    W_att_e = W_att[D:]

    BN = 1000
    BE = 2000
    NBT = N // BN
    NBQ = E // BE
    T, Q = pl.pallas_call(
        functools.partial(_tq_body, NBT),
        grid=(NBT + NBQ,),
        in_specs=[
            pl.BlockSpec((BN, D), lambda i: (jnp.minimum(i, NBT - 1), 0)),
            pl.BlockSpec((D, D), lambda i: (0, 0)),
            pl.BlockSpec((1, D), lambda i: (0, 0)),
            pl.BlockSpec((BE, DE), lambda i: (jnp.maximum(i - NBT, 0), 0)),
            pl.BlockSpec((DE, D), lambda i: (0, 0)),
        ],
        out_specs=[
            pl.BlockSpec((BN, D), lambda i: (jnp.minimum(i, NBT - 1), 0)),
            pl.BlockSpec((BE, D // 2), lambda i: (jnp.maximum(i - NBT, 0), 0)),
        ],
        out_shape=[
            jax.ShapeDtypeStruct((N, D), jnp.int32),
            jax.ShapeDtypeStruct((E, D // 2), jnp.int32),
        ],
    )(x, W_att_x, b_att.reshape(1, D), edge_attr, W_att_e)

    C = _CHUNK
    EPW = E // _NW
    NPHASE = _NPHASE
    CPP = EPW // C // NPHASE
    PAIRS = (CPP - 1) // 2
    RPT = ((N // _NS) + 7) // 8 * 8
    NPAD = RPT * _NS
    mesh = plsc.VectorSubcoreMesh(core_axis_name="c", subcore_axis_name="s")

    @functools.partial(
        pl.kernel,
        out_type=jax.ShapeDtypeStruct((_NC, NPAD, D), jnp.float32),
        mesh=mesh,
        scratch_types=[
            pltpu.VMEM((CPP, C), jnp.int32),
            pltpu.VMEM((CPP, C), jnp.int32),
            pltpu.VMEM((C, D), jnp.int32),
            pltpu.VMEM((C, D), jnp.int32),
            pltpu.VMEM((C, D // 2), jnp.int32),
            pltpu.VMEM((C, D // 2), jnp.int32),
            pltpu.VMEM((C, D), jnp.float32),
            pltpu.VMEM((C, D), jnp.float32),
            pltpu.MemorySpace.VMEM_SHARED((NPAD, D), jnp.float32),
            pltpu.SemaphoreType.DMA,
            pltpu.SemaphoreType.DMA,
            pltpu.SemaphoreType.DMA,
            pltpu.SemaphoreType.DMA,
            pltpu.SemaphoreType.DMA,
            pltpu.SemaphoreType.DMA,
        ],
    )
    def _sc_agg(t_hbm, q_hbm, col_hbm, row_hbm, z_hbm, out_hbm,
                colb, rowb, trA, trB, qA, qB, nA, nB, acc_sh,
                sga, sgb, sqa, sqb, ssa, ssb):
        cid = lax.axis_index("c")
        sid = lax.axis_index("s")
        wid = sid * _NC + cid
        pltpu.sync_copy(z_hbm, acc_sh.at[pl.ds(sid * RPT, RPT)])
        plsc.subcore_barrier()

        def issue(pbase, k, tr, q, sg, sq):
            pltpu.async_copy(t_hbm.at[colb.at[k]], tr, sg)
            pltpu.async_copy(q_hbm.at[pl.ds(pbase + k * C, C)], q, sq)

        def consume(pbase, k, tr, q, n, sg, sq, ss, wait_prev):
            pltpu.make_async_copy(t_hbm.at[colb.at[k]], tr, sg).wait()
            pltpu.make_async_copy(q_hbm.at[pl.ds(pbase + k * C, C)], q, sq).wait()

            def unpk(w):
                a = lax.bitcast_convert_type(
                    lax.shift_left(w, 16), jnp.float32)
                b = lax.bitcast_convert_type(
                    jnp.bitwise_and(w, jnp.int32(-65536)), jnp.float32)
                return a, b

            def sig_mul(xx, uu, vv):
                den = 1.0 + uu * vv
                seed = lax.bitcast_convert_type(
                    jnp.int32(0x7EF127EA)
                    - lax.bitcast_convert_type(den, jnp.int32),
                    jnp.float32,
                )
                r = seed * (2.0 - den * seed)
                r = r * (2.0 - den * r)
                return xx * r

            def sig_mul1(xx, uu, vv):
                den = 1.0 + uu * vv
                seed = lax.bitcast_convert_type(
                    jnp.int32(0x7EF127EA)
                    - lax.bitcast_convert_type(den, jnp.int32),
                    jnp.float32,
                )
                r = seed * (2.0 - den * seed)
                return xx * r

            def edge_body(e, carry2):
                h = D // 2
                for j in range(D // (2 * _L)):
                    xa, xb = unpk(tr[e, pl.ds(j * _L, _L)])
                    ua, ub = unpk(tr[e, pl.ds(h + j * _L, _L)])
                    va, vb = unpk(q[e, pl.ds(j * _L, _L)])
                    n[e, pl.ds(j * _L, _L)] = sig_mul1(xa, ua, va)
                    n[e, pl.ds(h + j * _L, _L)] = sig_mul1(xb, ub, vb)
                return carry2

            lax.fori_loop(0, C, edge_body, 0, unroll=8)
            pltpu.sync_copy(n, acc_sh.at[rowb.at[k]], add=True)

        def phase_body(p, carry):
            pltpu.sync_copy(col_hbm.at[wid, p], colb)
            pltpu.sync_copy(row_hbm.at[wid, p], rowb)
            pbase = (wid * NPHASE + p) * CPP * C
            issue(pbase, 0, trA, qA, sga, sqa)

            def pair_body(i, carry2):
                a = 2 * i
                issue(pbase, a + 1, trB, qB, sgb, sqb)
                consume(pbase, a, trA, qA, nA, sga, sqa, ssa, wait_prev=True)
                issue(pbase, a + 2, trA, qA, sga, sqa)
                consume(pbase, a + 1, trB, qB, nB, sgb, sqb, ssb, wait_prev=True)
                return carry2

            issue(pbase, 1, trB, qB, sgb, sqb)
            consume(pbase, 0, trA, qA, nA, sga, sqa, ssa, wait_prev=False)
            issue(pbase, 2, trA, qA, sga, sqa)
            consume(pbase, 1, trB, qB, nB, sgb, sqb, ssb, wait_prev=False)
            lax.fori_loop(1, PAIRS, pair_body, 0, unroll=False)
            consume(pbase, CPP - 1, trA, qA, nA, sga, sqa, ssa, wait_prev=True)
            return carry

        lax.fori_loop(0, NPHASE, phase_body, 0, unroll=False)

        plsc.subcore_barrier()
        pltpu.sync_copy(
            acc_sh.at[pl.ds(sid * RPT, RPT)],
            out_hbm.at[cid, pl.ds(sid * RPT, RPT)],
        )

    zeros = jnp.zeros((RPT, D), jnp.float32)
    col4 = col.reshape(_NW, NPHASE, CPP, C)
    row4 = row.reshape(_NW, NPHASE, CPP, C)
    agg2 = _sc_agg(T, Q, col4, row4, zeros)

    out = pl.pallas_call(
        _out_body,
        grid=(N // BN,),
        in_specs=[
            pl.BlockSpec((BN, D), lambda i: (i, 0)),
            pl.BlockSpec((1, BN, D), lambda i: (0, i, 0)),
            pl.BlockSpec((1, BN, D), lambda i: (1, i, 0)),
            pl.BlockSpec((D, D), lambda i: (0, 0)),
            pl.BlockSpec((D, D), lambda i: (0, 0)),
            pl.BlockSpec((1, D), lambda i: (0, 0)),
            pl.BlockSpec((1, D), lambda i: (0, 0)),
        ],
        out_specs=pl.BlockSpec((BN, D), lambda i: (i, 0)),
        out_shape=jax.ShapeDtypeStruct((N, D), jnp.float32),
    )(x, agg2, agg2, W_node, W_neigh, b_node.reshape(1, D), b_neigh.reshape(1, D))
    return out
